# Initial kernel scaffold; baseline (speedup 1.0000x reference)
import functools

import jax
import jax.numpy as jnp
from jax import lax
from jax.experimental import pallas as pl
from jax.experimental.pallas import tpu as pltpu
from jax.experimental.pallas import tpu_sc as plsc

B, L, V, D = 4096, 200, 32767, 128
NW = 32
RPW = B // NW
LH = L // 2
NV = D // 16


def _row_compute(buf, mask_v, bi, out_v):
    def body(l, carry):
        accs, den = carry[:NV], carry[NV]
        w = mask_v[bi, l]
        new = tuple(accs[d] + buf[l, pl.ds(16 * d, 16)] * w for d in range(NV))
        return new + (den + w,)

    init = tuple(jnp.zeros((16,), jnp.float32) for _ in range(NV))
    init = init + (jnp.float32(0.0),)
    res = lax.fori_loop(0, L, body, init)
    inv = 1.0 / jnp.maximum(res[NV], 1e-10)
    for d in range(NV):
        out_v[bi, pl.ds(16 * d, 16)] = res[d] * inv


def _bow_kernel(code_idx, code_mask, doc_idx, doc_mask, emb_code, emb_doc,
                out_code, out_doc,
                idx_v, mask_v, buf0, buf1, out_v, sem0, sem1):
    wid = lax.axis_index("s") * 2 + lax.axis_index("c")
    base = wid * RPW
    bufs = (buf0, buf1)
    sems = (sem0, sem1)

    def gather_row(table, bi, p):
        pltpu.async_copy(table.at[idx_v.at[bi, 0]],
                         bufs[p].at[pl.ds(0, LH)], sems[p])
        pltpu.async_copy(table.at[idx_v.at[bi, 1]],
                         bufs[p].at[pl.ds(LH, LH)], sems[p])

    def wait_row(table, p):
        pltpu.make_async_copy(table.at[pl.ds(0, L)], bufs[p], sems[p]).wait()

    for idx_hbm, mask_hbm, table, out_hbm in (
            (code_idx, code_mask, emb_code, out_code),
            (doc_idx, doc_mask, emb_doc, out_doc)):
        pltpu.sync_copy(idx_hbm.at[pl.ds(base, RPW)], idx_v)
        pltpu.sync_copy(mask_hbm.at[pl.ds(base, RPW)], mask_v)

        gather_row(table, 0, 0)
        gather_row(table, 1, 1)

        def step(g, _):
            for p in range(2):
                bi = g + p
                wait_row(table, p)
                _row_compute(bufs[p], mask_v, bi, out_v)
                gather_row(table, bi + 2, p)
            return 0

        pl.loop(0, RPW - 2, step=2)(step)

        for p in range(2):
            bi = RPW - 2 + p
            wait_row(table, p)
            _row_compute(bufs[p], mask_v, bi, out_v)

        pltpu.sync_copy(out_v, out_hbm.at[pl.ds(base, RPW)])


@jax.jit
def kernel(code_vec, code_mask, doc_vec, doc_mask, emb_code, emb_doc):
    code_idx = code_vec.astype(jnp.int32).reshape(B, 2, LH)
    doc_idx = doc_vec.astype(jnp.int32).reshape(B, 2, LH)
    run = functools.partial(
        pl.kernel,
        out_type=(jax.ShapeDtypeStruct((B, D), jnp.float32),
                  jax.ShapeDtypeStruct((B, D), jnp.float32)),
        mesh=plsc.VectorSubcoreMesh(core_axis_name="c", subcore_axis_name="s"),
        scratch_types=[
            pltpu.VMEM((RPW, 2, LH), jnp.int32),
            pltpu.VMEM((RPW, L), jnp.float32),
            pltpu.VMEM((L, D), jnp.float32),
            pltpu.VMEM((L, D), jnp.float32),
            pltpu.VMEM((RPW, D), jnp.float32),
            pltpu.SemaphoreType.DMA,
            pltpu.SemaphoreType.DMA,
        ],
    )(_bow_kernel)
    return run(code_idx, code_mask.astype(jnp.float32),
               doc_idx, doc_mask.astype(jnp.float32),
               emb_code, emb_doc)


# speedup vs baseline: 2.0201x; 2.0201x over previous
import functools

import jax
import jax.numpy as jnp
from jax import lax
from jax.experimental import pallas as pl
from jax.experimental.pallas import tpu as pltpu
from jax.experimental.pallas import tpu_sc as plsc

B, L, V, D = 4096, 200, 32767, 128
LP = 208
NW = 32
RPW = B // NW
LH = LP // 2
NV = D // 16
LB = LP // 16
CH = 32
NCH = RPW // CH


def _row_compute(buf, mask_v, bi, out_v, obi):
    def body(t, carry):
        accs, den = list(carry[:NV]), carry[NV]
        l0 = t * 16
        wv = mask_v[bi, pl.ds(l0, 16)]
        for j in range(16):
            w = wv[j]
            den = den + w
            for d in range(NV):
                accs[d] = accs[d] + buf[l0 + j, pl.ds(16 * d, 16)] * w
        return tuple(accs) + (den,)

    init = tuple(jnp.zeros((16,), jnp.float32) for _ in range(NV))
    init = init + (jnp.float32(0.0),)
    res = lax.fori_loop(0, LB, body, init)
    den_vec = jnp.broadcast_to(jnp.maximum(res[NV], 1e-10), (16,))
    inv = jnp.ones((16,), jnp.float32) / den_vec
    for d in range(NV):
        out_v[obi, pl.ds(16 * d, 16)] = res[d] * inv


def _bow_kernel(code_idx, code_mask, doc_idx, doc_mask, emb_code, emb_doc,
                out_code, out_doc,
                idx_v, mask_v, buf0, buf1, out_v, sem0, sem1):
    wid = lax.axis_index("s") * 2 + lax.axis_index("c")
    base = wid * RPW
    bufs = (buf0, buf1)
    sems = (sem0, sem1)

    def gather_row(table, bi, p):
        pltpu.async_copy(table.at[idx_v.at[bi, 0]],
                         bufs[p].at[pl.ds(0, LH)], sems[p])
        pltpu.async_copy(table.at[idx_v.at[bi, 1]],
                         bufs[p].at[pl.ds(LH, LH)], sems[p])

    def wait_row(table, p):
        pltpu.make_async_copy(table.at[pl.ds(0, LP)], bufs[p], sems[p]).wait()

    for idx_hbm, mask_hbm, table, out_hbm in (
            (code_idx, code_mask, emb_code, out_code),
            (doc_idx, doc_mask, emb_doc, out_doc)):

        def chunk_body(c):
            pltpu.sync_copy(idx_hbm.at[pl.ds(base + c * CH, CH)], idx_v)
            pltpu.sync_copy(mask_hbm.at[pl.ds(base + c * CH, CH)], mask_v)

            gather_row(table, 0, 0)
            gather_row(table, 1, 1)

            def step(g):
                for p in range(2):
                    bi = g + p
                    wait_row(table, p)
                    _row_compute(bufs[p], mask_v, bi, out_v, c * CH + bi)
                    gather_row(table, bi + 2, p)

            pl.loop(0, CH - 2, step=2)(step)

            for p in range(2):
                bi = CH - 2 + p
                wait_row(table, p)
                _row_compute(bufs[p], mask_v, bi, out_v, c * CH + bi)

        pl.loop(0, NCH)(chunk_body)
        pltpu.sync_copy(out_v, out_hbm.at[pl.ds(base, RPW)])


def _pad_idx(v):
    v = v.astype(jnp.int32)
    return jnp.pad(v, ((0, 0), (0, LP - L))).reshape(B, 2, LH)


def _pad_mask(m):
    return jnp.pad(m.astype(jnp.float32), ((0, 0), (0, LP - L)))


@jax.jit
def kernel(code_vec, code_mask, doc_vec, doc_mask, emb_code, emb_doc):
    run = functools.partial(
        pl.kernel,
        out_type=(jax.ShapeDtypeStruct((B, D), jnp.float32),
                  jax.ShapeDtypeStruct((B, D), jnp.float32)),
        mesh=plsc.VectorSubcoreMesh(core_axis_name="c", subcore_axis_name="s"),
        scratch_types=[
            pltpu.VMEM((CH, 2, LH), jnp.int32),
            pltpu.VMEM((CH, LP), jnp.float32),
            pltpu.VMEM((LP, D), jnp.float32),
            pltpu.VMEM((LP, D), jnp.float32),
            pltpu.VMEM((RPW, D), jnp.float32),
            pltpu.SemaphoreType.DMA,
            pltpu.SemaphoreType.DMA,
        ],
    )(_bow_kernel)
    return run(_pad_idx(code_vec), _pad_mask(code_mask),
               _pad_idx(doc_vec), _pad_mask(doc_mask),
               emb_code, emb_doc)


# speedup vs baseline: 2.0208x; 1.0003x over previous
import functools

import jax
import jax.numpy as jnp
import numpy as np
from jax import lax
from jax.experimental import pallas as pl
from jax.experimental.pallas import tpu as pltpu
from jax.experimental.pallas import tpu_sc as plsc

B, L, V, D = 4096, 200, 32767, 128
LP = 208
NW = 32
RPW = B // NW
LH = LP // 2
NV = D // 16
LB = LP // 16
CH = 32
NCH = RPW // CH


def _row_compute(buf, mask_v, bi, out_v, obi, lane_j, lane_rot):
    def body(t, carry):
        accs, den = list(carry[:NV]), carry[NV]
        l0 = t * 16
        wv = mask_v[bi, pl.ds(l0, 16)]
        den = den + wv
        for j in range(16):
            wj = wv.at[lane_j[j]].get(mode="promise_in_bounds")
            for d in range(NV):
                accs[d] = accs[d] + buf[l0 + j, pl.ds(16 * d, 16)] * wj
        return tuple(accs) + (den,)

    init = tuple(jnp.zeros((16,), jnp.float32) for _ in range(NV + 1))
    res = lax.fori_loop(0, LB, body, init)
    den = res[NV]
    for rot in lane_rot:
        den = den + den.at[rot].get(mode="promise_in_bounds")
    inv = jnp.ones((16,), jnp.float32) / jnp.maximum(den, 1e-10)
    for d in range(NV):
        out_v[obi, pl.ds(16 * d, 16)] = res[d] * inv


def _bow_kernel(code_idx, code_mask, doc_idx, doc_mask, emb_code, emb_doc,
                out_code, out_doc,
                idx_v, mask_v, buf0, buf1, out_v, sem0, sem1):
    wid = lax.axis_index("s") * 2 + lax.axis_index("c")
    base = wid * RPW
    bufs = (buf0, buf1)
    sems = (sem0, sem1)
    lanes = lax.iota(jnp.int32, 16)
    lane_j = [lanes * 0 + j for j in range(16)]
    lane_rot = [(lanes + s) & 15 for s in (1, 2, 4, 8)]

    def gather_row(table, bi, p):
        pltpu.async_copy(table.at[idx_v.at[bi, 0]],
                         bufs[p].at[pl.ds(0, LH)], sems[p])
        pltpu.async_copy(table.at[idx_v.at[bi, 1]],
                         bufs[p].at[pl.ds(LH, LH)], sems[p])

    def wait_row(table, p):
        pltpu.make_async_copy(table.at[pl.ds(0, LP)], bufs[p], sems[p]).wait()

    for idx_hbm, mask_hbm, table, out_hbm in (
            (code_idx, code_mask, emb_code, out_code),
            (doc_idx, doc_mask, emb_doc, out_doc)):

        def chunk_body(c):
            pltpu.sync_copy(idx_hbm.at[pl.ds(base + c * CH, CH)], idx_v)
            pltpu.sync_copy(mask_hbm.at[pl.ds(base + c * CH, CH)], mask_v)

            gather_row(table, 0, 0)
            gather_row(table, 1, 1)

            def step(g):
                for p in range(2):
                    bi = g + p
                    wait_row(table, p)
                    _row_compute(bufs[p], mask_v, bi, out_v, c * CH + bi, lane_j, lane_rot)
                    gather_row(table, bi + 2, p)

            pl.loop(0, CH - 2, step=2)(step)

            for p in range(2):
                bi = CH - 2 + p
                wait_row(table, p)
                _row_compute(bufs[p], mask_v, bi, out_v, c * CH + bi, lane_j, lane_rot)

        pl.loop(0, NCH)(chunk_body)
        pltpu.sync_copy(out_v, out_hbm.at[pl.ds(base, RPW)])


def _pad_idx(v):
    v = v.astype(jnp.int32)
    return jnp.pad(v, ((0, 0), (0, LP - L))).reshape(B, 2, LH)


def _pad_mask(m):
    return jnp.pad(m.astype(jnp.float32), ((0, 0), (0, LP - L)))


@jax.jit
def kernel(code_vec, code_mask, doc_vec, doc_mask, emb_code, emb_doc):
    run = functools.partial(
        pl.kernel,
        out_type=(jax.ShapeDtypeStruct((B, D), jnp.float32),
                  jax.ShapeDtypeStruct((B, D), jnp.float32)),
        mesh=plsc.VectorSubcoreMesh(core_axis_name="c", subcore_axis_name="s"),
        scratch_types=[
            pltpu.VMEM((CH, 2, LH), jnp.int32),
            pltpu.VMEM((CH, LP), jnp.float32),
            pltpu.VMEM((LP, D), jnp.float32),
            pltpu.VMEM((LP, D), jnp.float32),
            pltpu.VMEM((RPW, D), jnp.float32),
            pltpu.SemaphoreType.DMA,
            pltpu.SemaphoreType.DMA,
        ],
    )(_bow_kernel)
    return run(_pad_idx(code_vec), _pad_mask(code_mask),
               _pad_idx(doc_vec), _pad_mask(doc_mask),
               emb_code, emb_doc)
